# baseline (speedup 1.0000x reference)
import functools
import math

import jax
import jax.numpy as jnp
from jax import lax
from jax.experimental import pallas as pl
from jax.experimental.pallas import tpu as pltpu
from jax.experimental.pallas import tpu_sc as plsc

D_EMBED = 64
SCALE = math.sqrt(D_EMBED)

NC = 2
NS = 16
NW = NC * NS

TOK_PER_W = 25600
SEG_ROWS = 40
SEG_TOK = SEG_ROWS * 128
NSEG = TOK_PER_W // SEG_TOK
CT = 256
NCH = SEG_TOK // CT
LANES = 16


def _lookup(idx2d, t2):
    B = idx2d.shape[0] * idx2d.shape[1]

    mesh = plsc.VectorSubcoreMesh(core_axis_name="c", subcore_axis_name="s")

    @functools.partial(
        pl.kernel,
        out_type=jax.ShapeDtypeStruct((B, D_EMBED), jnp.float32),
        mesh=mesh,
        scratch_types=[
            pltpu.VMEM((SEG_ROWS, 128), jnp.int32),
            pltpu.VMEM((SEG_TOK,), jnp.int32),
            pltpu.VMEM((SEG_TOK,), jnp.int32),
            pltpu.VMEM((CT, 128), jnp.float32),
            pltpu.VMEM((CT, D_EMBED), jnp.float32),
            pltpu.SemaphoreType.DMA,
        ],
        compiler_params=pltpu.CompilerParams(needs_layout_passes=False),
    )
    def body(idx_hbm, tab_hbm, out_hbm, staging, hv, phys, rows, outv, gsem):
        wid = lax.axis_index("s") * NC + lax.axis_index("c")
        row0 = 200 * wid

        @pl.loop(0, NSEG)
        def _seg(s):
            r0 = pl.multiple_of(row0 + SEG_ROWS * s, 8)
            pltpu.sync_copy(idx_hbm.at[pl.ds(r0, SEG_ROWS)], staging)

            @pl.loop(0, SEG_ROWS)
            def _tr(r):
                for j in range(8):
                    v = staging[r, pl.ds(LANES * j, LANES)]
                    base = r * 128 + LANES * j
                    phys[pl.ds(base, LANES)] = v >> 1
                    hv[pl.ds(base, LANES)] = (v & 1) << 6

            @pl.loop(0, NCH)
            def _ch(c):
                t0 = c * CT
                cps = [
                    pltpu.async_copy(
                        tab_hbm.at[phys.at[pl.ds(t0 + 128 * g, 128)]],
                        rows.at[pl.ds(128 * g, 128)],
                        gsem,
                    )
                    for g in range(CT // 128)
                ]
                for cp in cps:
                    cp.wait()

                iota = lax.broadcasted_iota(jnp.int32, (LANES,), 0)

                @pl.loop(0, CT // LANES)
                def _sc(k):
                    tok = iota + k * LANES
                    hvec = hv[pl.ds(t0 + k * LANES, LANES)]
                    for j in range(D_EMBED):
                        vals = plsc.load_gather(rows, [tok, hvec + j])
                        plsc.store_scatter(
                            outv, [tok, iota * 0 + j], vals * SCALE
                        )

                off = TOK_PER_W * wid + SEG_TOK * s + t0
                pltpu.sync_copy(outv, out_hbm.at[pl.ds(off, CT)])

    return body(idx2d, t2)


def kernel(x, table):
    B = x.shape[0] * x.shape[1]
    idx2d = x.reshape(B // 128, 128).astype(jnp.int32)
    t2 = table.reshape(table.shape[0] // 2, 128)
    out = _lookup(idx2d, t2)
    return out.reshape(x.shape[0], x.shape[1], D_EMBED)


# speedup vs baseline: 2.0367x; 2.0367x over previous
import functools
import math

import jax
import jax.numpy as jnp
from jax import lax
from jax.experimental import pallas as pl
from jax.experimental.pallas import tpu as pltpu
from jax.experimental.pallas import tpu_sc as plsc

D_EMBED = 64
SCALE = math.sqrt(D_EMBED)

NC = 2
NS = 16
NW = NC * NS

TOK_PER_W = 25600
CHUNK = 512
NCH = TOK_PER_W // CHUNK
LANES = 16

KB_W = 8192
KB_N = -(-1000000 // KB_W)


def _expand_table(tT):
    V = tT.shape[1]

    def body(t_ref, o_ref):
        tt = jnp.transpose(t_ref[...])
        o_ref[...] = jnp.concatenate([tt, tt], axis=1)

    return pl.pallas_call(
        body,
        grid=(KB_N,),
        in_specs=[pl.BlockSpec((tT.shape[0], KB_W), lambda i: (0, i))],
        out_specs=pl.BlockSpec((KB_W, 128), lambda i: (i, 0)),
        out_shape=jax.ShapeDtypeStruct((V, 128), jnp.float32),
    )(tT)


def _gather_scale(idx2d, t2):
    B = idx2d.shape[0] * idx2d.shape[1]
    rows_per_w = TOK_PER_W // 128

    mesh = plsc.VectorSubcoreMesh(core_axis_name="c", subcore_axis_name="s")

    @functools.partial(
        pl.kernel,
        out_type=jax.ShapeDtypeStruct((B // 2, 128), jnp.float32),
        mesh=mesh,
        scratch_types=[
            pltpu.VMEM((rows_per_w, 128), jnp.int32),
            pltpu.VMEM((CHUNK, 128), jnp.float32),
            pltpu.VMEM((CHUNK // 2, 128), jnp.float32),
            pltpu.SemaphoreType.DMA,
        ],
    )
    def body(idx_hbm, tab_hbm, out_hbm, idx_all, rows, outv, gsem):
        wid = lax.axis_index("s") * NC + lax.axis_index("c")
        ibase = pl.multiple_of(wid * rows_per_w, 8)
        base = wid * TOK_PER_W

        pltpu.sync_copy(idx_hbm.at[pl.ds(ibase, rows_per_w)], idx_all)

        @pl.loop(0, NCH)
        def _ch(c):
            cps = [
                pltpu.async_copy(
                    tab_hbm.at[idx_all.at[(CHUNK // 128) * c + g]],
                    rows.at[pl.ds(128 * g, 128)],
                    gsem,
                )
                for g in range(CHUNK // 128)
            ]
            for cp in cps:
                cp.wait()

            @pl.loop(0, CHUNK // 2, unroll=4)
            def _sc(p):
                for h in range(2):
                    for j in range(D_EMBED // LANES):
                        outv[p, pl.ds(64 * h + LANES * j, LANES)] = (
                            rows[2 * p + h, pl.ds(LANES * j, LANES)] * SCALE
                        )

            woff = pl.multiple_of((base + CHUNK * c) // 2, 8)
            pltpu.sync_copy(outv, out_hbm.at[pl.ds(woff, CHUNK // 2)])

    return body(idx2d, t2)


def kernel(x, table):
    B = x.shape[0] * x.shape[1]
    idx2d = x.reshape(B // 128, 128).astype(jnp.int32)
    t2 = _expand_table(table.T)
    out = _gather_scale(idx2d, t2)
    return out.reshape(x.shape[0], x.shape[1], D_EMBED)


# speedup vs baseline: 2.8078x; 1.3786x over previous
import functools
import math

import jax
import jax.numpy as jnp
from jax import lax
from jax.experimental import pallas as pl
from jax.experimental.pallas import tpu as pltpu
from jax.experimental.pallas import tpu_sc as plsc

D_EMBED = 64
SCALE = math.sqrt(D_EMBED)

NC = 2
NS = 16
NW = NC * NS

TOK_PER_W = 25600
CHUNK = 256
NCH = TOK_PER_W // CHUNK
LANES = 16

KB_W = 8192
KB_N = -(-1000000 // KB_W)


def _expand_table(tT):
    V = tT.shape[1]

    def body(t_ref, o_ref):
        tt = jnp.transpose(t_ref[...])
        o_ref[...] = jnp.concatenate([tt, tt], axis=1)

    return pl.pallas_call(
        body,
        grid=(KB_N,),
        in_specs=[pl.BlockSpec((tT.shape[0], KB_W), lambda i: (0, i))],
        out_specs=pl.BlockSpec((KB_W, 128), lambda i: (i, 0)),
        out_shape=jax.ShapeDtypeStruct((V, 128), jnp.float32),
    )(tT)


def _gather_scale(idx2d, t2):
    B = idx2d.shape[0] * idx2d.shape[1]
    rows_per_w = TOK_PER_W // 128

    mesh = plsc.VectorSubcoreMesh(core_axis_name="c", subcore_axis_name="s")

    @functools.partial(
        pl.kernel,
        out_type=jax.ShapeDtypeStruct((B, D_EMBED), jnp.float32),
        mesh=mesh,
        scratch_types=[
            pltpu.VMEM((rows_per_w, 128), jnp.int32),
            pltpu.VMEM((CHUNK, 128), jnp.float32),
            pltpu.VMEM((CHUNK, 128), jnp.float32),
            pltpu.VMEM((CHUNK, D_EMBED), jnp.float32),
            pltpu.SemaphoreType.DMA,
            pltpu.SemaphoreType.DMA,
        ],
    )
    def body(idx_hbm, tab_hbm, out_hbm, idx_all, rows_a, rows_b, outv,
             sem_a, sem_b):
        wid = lax.axis_index("s") * NC + lax.axis_index("c")
        ibase = pl.multiple_of(wid * rows_per_w, 8)
        base = wid * TOK_PER_W
        ng = CHUNK // 128

        pltpu.sync_copy(idx_hbm.at[pl.ds(ibase, rows_per_w)], idx_all)

        def fire(c, buf, sem):
            return [
                pltpu.async_copy(
                    tab_hbm.at[idx_all.at[ng * c + g]],
                    buf.at[pl.ds(128 * g, 128)],
                    sem,
                )
                for g in range(ng)
            ]

        def finish(c, buf, cps):
            for cp in cps:
                cp.wait()
            @pl.loop(0, CHUNK, unroll=8)
            def _sc(t):
                for j in range(D_EMBED // LANES):
                    sl = pl.ds(LANES * j, LANES)
                    outv[t, sl] = buf[t, sl] * SCALE

            woff = pl.multiple_of(base + CHUNK * c, 8)
            pltpu.sync_copy(outv, out_hbm.at[pl.ds(woff, CHUNK)])

        cps0 = fire(0, rows_a, sem_a)

        @pl.loop(0, NCH // 2)
        def _pair(g):
            a = 2 * g
            cps_b = fire(a + 1, rows_b, sem_b)
            finish(a, rows_a, cps0)

            @pl.when(g < NCH // 2 - 1)
            def _():
                fire(a + 2, rows_a, sem_a)

            finish(a + 1, rows_b, cps_b)

    return body(idx2d, t2)


def kernel(x, table):
    B = x.shape[0] * x.shape[1]
    idx2d = x.reshape(B // 128, 128).astype(jnp.int32)
    t2 = _expand_table(table.T)
    out = _gather_scale(idx2d, t2)
    return out.reshape(x.shape[0], x.shape[1], D_EMBED)


# speedup vs baseline: 2.8810x; 1.0261x over previous
import functools
import math

import jax
import jax.numpy as jnp
from jax import lax
from jax.experimental import pallas as pl
from jax.experimental.pallas import tpu as pltpu
from jax.experimental.pallas import tpu_sc as plsc

D_EMBED = 64
SCALE = math.sqrt(D_EMBED)

NC = 2
NS = 16
NW = NC * NS

TOK_PER_W = 25600
CHUNK = 128
NCH = TOK_PER_W // CHUNK
DEPTH = 4
LANES = 16

KB_W = 16384
KB_N = -(-1000000 // KB_W)


def _expand_table(tT):
    V = tT.shape[1]

    def body(t_ref, o_ref):
        tt = jnp.transpose(t_ref[...])
        o_ref[...] = jnp.concatenate([tt, tt], axis=1)

    return pl.pallas_call(
        body,
        grid=(KB_N,),
        in_specs=[pl.BlockSpec((tT.shape[0], KB_W), lambda i: (0, i))],
        out_specs=pl.BlockSpec((KB_W, 128), lambda i: (i, 0)),
        out_shape=jax.ShapeDtypeStruct((V, 128), jnp.float32),
    )(tT)


def _gather_scale(idx2d, t2):
    B = idx2d.shape[0] * idx2d.shape[1]
    rows_per_w = TOK_PER_W // 128

    mesh = plsc.VectorSubcoreMesh(core_axis_name="c", subcore_axis_name="s")

    @functools.partial(
        pl.kernel,
        out_type=jax.ShapeDtypeStruct((B, D_EMBED), jnp.float32),
        mesh=mesh,
        scratch_types=[
            pltpu.VMEM((rows_per_w, 128), jnp.int32),
            [pltpu.VMEM((CHUNK, 128), jnp.float32) for _ in range(DEPTH)],
            pltpu.VMEM((CHUNK, D_EMBED), jnp.float32),
            [pltpu.SemaphoreType.DMA for _ in range(DEPTH)],
        ],
    )
    def body(idx_hbm, tab_hbm, out_hbm, idx_all, rows, outv, sems):
        wid = lax.axis_index("s") * NC + lax.axis_index("c")
        ibase = pl.multiple_of(wid * rows_per_w, 8)
        base = wid * TOK_PER_W

        pltpu.sync_copy(idx_hbm.at[pl.ds(ibase, rows_per_w)], idx_all)

        def fire(c, b):
            pltpu.async_copy(tab_hbm.at[idx_all.at[c]], rows[b], sems[b])

        def finish(c, b):
            pltpu.make_async_copy(
                tab_hbm.at[pl.ds(0, CHUNK)], rows[b], sems[b]
            ).wait()

            @pl.loop(0, CHUNK, unroll=8)
            def _sc(t):
                for j in range(D_EMBED // LANES):
                    sl = pl.ds(LANES * j, LANES)
                    outv[t, sl] = rows[b][t, sl] * SCALE

            woff = pl.multiple_of(base + CHUNK * c, 8)
            pltpu.sync_copy(outv, out_hbm.at[pl.ds(woff, CHUNK)])

        for c in range(DEPTH - 1):
            fire(c, c)

        @pl.loop(0, NCH // DEPTH)
        def _quad(q):
            c0 = DEPTH * q
            for b in range(DEPTH):
                c = c0 + b
                nxt = (b + DEPTH - 1) % DEPTH
                @pl.when(c + DEPTH - 1 < NCH)
                def _():
                    fire(c + DEPTH - 1, nxt)
                finish(c, b)

    return body(idx2d, t2)


def kernel(x, table):
    B = x.shape[0] * x.shape[1]
    idx2d = x.reshape(B // 128, 128).astype(jnp.int32)
    t2 = _expand_table(table.T)
    out = _gather_scale(idx2d, t2)
    return out.reshape(x.shape[0], x.shape[1], D_EMBED)


# speedup vs baseline: 3.2452x; 1.1264x over previous
import functools
import math

import jax
import jax.numpy as jnp
from jax import lax
from jax.experimental import pallas as pl
from jax.experimental.pallas import tpu as pltpu
from jax.experimental.pallas import tpu_sc as plsc

D_EMBED = 64
SCALE = math.sqrt(D_EMBED)

NC = 2
NS = 16
NW = NC * NS

TOK_PER_W = 25600
CHUNK = 128
NCH = TOK_PER_W // CHUNK
DEPTH = 4
LANES = 16

KB_W = 16384
KB_N = -(-1000000 // KB_W)


def _expand_table(tT):
    V = tT.shape[1]

    def body(t_ref, o_ref):
        tt = jnp.transpose(t_ref[...])
        o_ref[...] = jnp.concatenate([tt, tt], axis=1)

    return pl.pallas_call(
        body,
        grid=(KB_N,),
        in_specs=[pl.BlockSpec((tT.shape[0], KB_W), lambda i: (0, i))],
        out_specs=pl.BlockSpec((KB_W, 128), lambda i: (i, 0)),
        out_shape=jax.ShapeDtypeStruct((V, 128), jnp.float32),
    )(tT)


def _gather_scale(idx2d, t2):
    B = idx2d.shape[0] * idx2d.shape[1]
    rows_per_w = TOK_PER_W // 128

    mesh = plsc.VectorSubcoreMesh(core_axis_name="c", subcore_axis_name="s")

    @functools.partial(
        pl.kernel,
        out_type=jax.ShapeDtypeStruct((B, D_EMBED), jnp.float32),
        mesh=mesh,
        scratch_types=[
            pltpu.VMEM((rows_per_w, 128), jnp.int32),
            [pltpu.VMEM((CHUNK, 128), jnp.float32) for _ in range(DEPTH)],
            [pltpu.VMEM((CHUNK, D_EMBED), jnp.float32) for _ in range(2)],
            [pltpu.SemaphoreType.DMA for _ in range(DEPTH)],
            [pltpu.SemaphoreType.DMA for _ in range(2)],
        ],
    )
    def body(idx_hbm, tab_hbm, out_hbm, idx_all, rows, outv, sems, wsems):
        wid = lax.axis_index("s") * NC + lax.axis_index("c")
        ibase = pl.multiple_of(wid * rows_per_w, 8)
        base = wid * TOK_PER_W

        pltpu.sync_copy(idx_hbm.at[pl.ds(ibase, rows_per_w)], idx_all)

        def fire(c, b):
            pltpu.async_copy(tab_hbm.at[idx_all.at[c]], rows[b], sems[b])

        def wb_wait(o):
            pltpu.make_async_copy(
                outv[o], out_hbm.at[pl.ds(0, CHUNK)], wsems[o]
            ).wait()

        def finish(c, b, o, first_round):
            pltpu.make_async_copy(
                tab_hbm.at[pl.ds(0, CHUNK)], rows[b], sems[b]
            ).wait()
            if first_round:
                @pl.when(c >= 2)
                def _():
                    wb_wait(o)
            else:
                wb_wait(o)

            @pl.loop(0, CHUNK, unroll=8)
            def _sc(t):
                for j in range(D_EMBED // LANES):
                    sl = pl.ds(LANES * j, LANES)
                    outv[o][t, sl] = rows[b][t, sl] * SCALE

            woff = pl.multiple_of(base + CHUNK * c, 8)
            pltpu.async_copy(outv[o], out_hbm.at[pl.ds(woff, CHUNK)], wsems[o])

        for c in range(DEPTH - 1):
            fire(c, c)

        def round_of(q, first_round):
            c0 = DEPTH * q
            for b in range(DEPTH):
                c = c0 + b
                nxt = (b + DEPTH - 1) % DEPTH
                @pl.when(c + DEPTH - 1 < NCH)
                def _():
                    fire(c + DEPTH - 1, nxt)
                finish(c, b, (b % 2), first_round)

        round_of(0, True)

        @pl.loop(1, NCH // DEPTH)
        def _quad(q):
            round_of(q, False)

        wb_wait(0)
        wb_wait(1)

    return body(idx2d, t2)


def kernel(x, table):
    B = x.shape[0] * x.shape[1]
    idx2d = x.reshape(B // 128, 128).astype(jnp.int32)
    t2 = _expand_table(table.T)
    out = _gather_scale(idx2d, t2)
    return out.reshape(x.shape[0], x.shape[1], D_EMBED)
